# Initial kernel scaffold; baseline (speedup 1.0000x reference)
import jax
import jax.numpy as jnp
from jax.experimental import pallas as pl


def kernel(graph, batch, entity_emb, relation_emb):
    raise NotImplementedError("write your pallas kernel here")


# speedup vs baseline: 1.3698x; 1.3698x over previous
import functools

import jax
import jax.numpy as jnp
from jax import lax
from jax.experimental import pallas as pl
from jax.experimental.pallas import tpu as pltpu
from jax.experimental.pallas import tpu_sc as plsc

_NC = 2
_NS = 16
_NW = _NC * _NS
_D = 64
_L = 16


def _build_sc_kernel(n_items):
    per_w = n_items // _NW
    K = 256
    n_chunks = per_w // K

    @functools.partial(
        pl.kernel,
        out_type=jax.ShapeDtypeStruct((n_items,), jnp.float32),
        mesh=plsc.VectorSubcoreMesh(core_axis_name="c", subcore_axis_name="s"),
        compiler_params=pltpu.CompilerParams(
            needs_layout_passes=False, use_tc_tiling_on_sc=False),
        scratch_types=[
            pltpu.VMEM((K,), jnp.int32),
            pltpu.VMEM((K,), jnp.int32),
            pltpu.VMEM((K,), jnp.int32),
            pltpu.VMEM((K, _D), jnp.float32),
            pltpu.VMEM((K, _D), jnp.float32),
            pltpu.VMEM((K, _D), jnp.float32),
            pltpu.VMEM((K,), jnp.float32),
            pltpu.SemaphoreType.DMA,
            pltpu.SemaphoreType.DMA,
            pltpu.SemaphoreType.DMA,
        ],
    )
    def sc_k(hidx_hbm, tidx_hbm, ridx_hbm, ent_hbm, rel_hbm, out_hbm,
             hidx_v, tidx_v, ridx_v, hrow_v, trow_v, rrow_v, out_v,
             sem_h, sem_t, sem_r):
        wid = lax.axis_index("s") * _NC + lax.axis_index("c")
        wbase = wid * per_w

        def chunk_body(c, carry):
            base = wbase + c * K
            pltpu.sync_copy(hidx_hbm.at[pl.ds(base, K)], hidx_v)
            pltpu.sync_copy(tidx_hbm.at[pl.ds(base, K)], tidx_v)
            pltpu.sync_copy(ridx_hbm.at[pl.ds(base, K)], ridx_v)
            cph = pltpu.async_copy(ent_hbm.at[hidx_v], hrow_v, sem_h)
            cpt = pltpu.async_copy(ent_hbm.at[tidx_v], trow_v, sem_t)
            cpr = pltpu.async_copy(rel_hbm.at[ridx_v], rrow_v, sem_r)
            cph.wait()
            cpt.wait()
            cpr.wait()

            lanes = lax.iota(jnp.int32, _L)

            def item_body(j, carry2):
                ids = j * _L + lanes
                acc = jnp.zeros((_L,), jnp.float32)
                for d in range(_D):
                    col = jnp.full((_L,), d, jnp.int32)
                    hv = plsc.load_gather(hrow_v, [ids, col])
                    tv = plsc.load_gather(trow_v, [ids, col])
                    rv = plsc.load_gather(rrow_v, [ids, col])
                    acc = acc + hv * tv * rv
                out_v[pl.ds(j * _L, _L)] = acc
                return carry2

            lax.fori_loop(0, K // _L, item_body, 0)
            pltpu.sync_copy(out_v, out_hbm.at[pl.ds(base, K)])
            return carry

        lax.fori_loop(0, n_chunks, chunk_body, 0)

    return sc_k


def kernel(graph, batch, entity_emb, relation_emb):
    B, C, _ = batch.shape
    n = B * C
    h_idx = batch[..., 0].reshape(n).astype(jnp.int32)
    t_idx = batch[..., 1].reshape(n).astype(jnp.int32)
    r_idx = batch[..., 2].reshape(n).astype(jnp.int32)
    sc_k = _build_sc_kernel(n)
    out = sc_k(h_idx, t_idx, r_idx, entity_emb, relation_emb)
    return out.reshape(B, C)


# speedup vs baseline: 1.3946x; 1.0181x over previous
import functools

import jax
import jax.numpy as jnp
from jax import lax
from jax.experimental import pallas as pl
from jax.experimental.pallas import tpu as pltpu
from jax.experimental.pallas import tpu_sc as plsc

_NC = 2
_NS = 16
_NW = _NC * _NS
_D = 64
_L = 16


def _build_sc_kernel(n_items):
    per_w = n_items // _NW
    K = 256
    n_chunks = per_w // K
    half = (n_chunks - 1) // 2

    row_buf = pltpu.VMEM((K, _D), jnp.float32)
    idx_buf = pltpu.VMEM((K,), jnp.int32)

    @functools.partial(
        pl.kernel,
        out_type=jax.ShapeDtypeStruct((n_items,), jnp.float32),
        mesh=plsc.VectorSubcoreMesh(core_axis_name="c", subcore_axis_name="s"),
        compiler_params=pltpu.CompilerParams(
            needs_layout_passes=False, use_tc_tiling_on_sc=False),
        scratch_types=[
            pltpu.VMEM((3 * K,), jnp.int32),
            pltpu.VMEM((3 * K,), jnp.int32),
            idx_buf, idx_buf, idx_buf,
            idx_buf, idx_buf, idx_buf,
            row_buf, row_buf, row_buf,
            row_buf, row_buf, row_buf,
            pltpu.VMEM((K,), jnp.float32),
            pltpu.SemaphoreType.DMA, pltpu.SemaphoreType.DMA,
            pltpu.SemaphoreType.DMA, pltpu.SemaphoreType.DMA,
            pltpu.SemaphoreType.DMA, pltpu.SemaphoreType.DMA,
        ],
    )
    def sc_k(batch_hbm, ent_hbm, rel_hbm, out_hbm,
             bidx_a, bidx_b,
             hidx_a, tidx_a, ridx_a, hidx_b, tidx_b, ridx_b,
             hrow_a, trow_a, rrow_a, hrow_b, trow_b, rrow_b,
             out_v,
             sh_a, st_a, sr_a, sh_b, st_b, sr_b):
        wid = lax.axis_index("s") * _NC + lax.axis_index("c")
        wbase = wid * per_w
        lanes = lax.iota(jnp.int32, _L)
        lanes3 = lanes * 3
        lanes_row = lanes * _D

        bufs_a = (bidx_a, hidx_a, tidx_a, ridx_a, hrow_a, trow_a, rrow_a,
                  sh_a, st_a, sr_a)
        bufs_b = (bidx_b, hidx_b, tidx_b, ridx_b, hrow_b, trow_b, rrow_b,
                  sh_b, st_b, sr_b)

        def issue(c, bufs):
            bidx_v, hidx_v, tidx_v, ridx_v, hrow_v, trow_v, rrow_v, sh, st, sr = bufs
            base = wbase + c * K
            pltpu.sync_copy(batch_hbm.at[pl.ds(base * 3, 3 * K)], bidx_v)
            for g in range(K // _L):
                b0 = g * (3 * _L)
                hidx_v[pl.ds(g * _L, _L)] = plsc.load_gather(
                    bidx_v, [b0 + lanes3])
                tidx_v[pl.ds(g * _L, _L)] = plsc.load_gather(
                    bidx_v, [b0 + lanes3 + 1])
                ridx_v[pl.ds(g * _L, _L)] = plsc.load_gather(
                    bidx_v, [b0 + lanes3 + 2])
            pltpu.async_copy(ent_hbm.at[hidx_v], hrow_v, sh)
            pltpu.async_copy(ent_hbm.at[tidx_v], trow_v, st)
            pltpu.async_copy(rel_hbm.at[ridx_v], rrow_v, sr)

        def wait_and_compute(c, bufs):
            _, hidx_v, tidx_v, ridx_v, hrow_v, trow_v, rrow_v, sh, st, sr = bufs
            base = wbase + c * K
            pltpu.make_async_copy(ent_hbm.at[hidx_v], hrow_v, sh).wait()
            pltpu.make_async_copy(ent_hbm.at[tidx_v], trow_v, st).wait()
            pltpu.make_async_copy(rel_hbm.at[ridx_v], rrow_v, sr).wait()
            def group_body(g, carry):
                ids = g * _L + lanes
                acc = jnp.zeros((_L,), jnp.float32)
                for d in range(_D):
                    col = jnp.full((_L,), d, jnp.int32)
                    hv = plsc.load_gather(hrow_v, [ids, col])
                    tv = plsc.load_gather(trow_v, [ids, col])
                    rv = plsc.load_gather(rrow_v, [ids, col])
                    acc = acc + hv * tv * rv
                out_v[pl.ds(g * _L, _L)] = acc
                return carry

            lax.fori_loop(0, K // _L, group_body, 0)
            pltpu.sync_copy(out_v, out_hbm.at[pl.ds(base, K)])

        issue(0, bufs_a)

        def body2(c2, carry):
            c = 2 * c2
            issue(c + 1, bufs_b)
            wait_and_compute(c, bufs_a)
            issue(c + 2, bufs_a)
            wait_and_compute(c + 1, bufs_b)
            return carry

        lax.fori_loop(0, half, body2, 0)
        wait_and_compute(n_chunks - 1, bufs_a)

    return sc_k


def kernel(graph, batch, entity_emb, relation_emb):
    B, C, _ = batch.shape
    n = B * C
    flat = batch.reshape(n * 3).astype(jnp.int32)
    sc_k = _build_sc_kernel(n)
    out = sc_k(flat, entity_emb, relation_emb)
    return out.reshape(B, C)


# speedup vs baseline: 2.1459x; 1.5388x over previous
import functools

import jax
import jax.numpy as jnp
from jax import lax
from jax.experimental import pallas as pl
from jax.experimental.pallas import tpu as pltpu
from jax.experimental.pallas import tpu_sc as plsc

_NC = 2
_NS = 16
_NW = _NC * _NS
_D = 64
_L = 16


def _build_sc_kernel(n_items):
    per_w = n_items // _NW
    K = 256
    n_chunks = per_w // K
    half = (n_chunks - 1) // 2

    row_buf = pltpu.VMEM((K, _D), jnp.float32)
    idx_buf = pltpu.VMEM((K,), jnp.int32)

    @functools.partial(
        pl.kernel,
        out_type=jax.ShapeDtypeStruct((n_items,), jnp.float32),
        mesh=plsc.VectorSubcoreMesh(core_axis_name="c", subcore_axis_name="s"),
        compiler_params=pltpu.CompilerParams(
            needs_layout_passes=False, use_tc_tiling_on_sc=False),
        scratch_types=[
            pltpu.VMEM((3 * K,), jnp.int32),
            pltpu.VMEM((3 * K,), jnp.int32),
            idx_buf, idx_buf, idx_buf,
            idx_buf, idx_buf, idx_buf,
            row_buf, row_buf, row_buf,
            row_buf, row_buf, row_buf,
            pltpu.VMEM((K,), jnp.float32),
            pltpu.SemaphoreType.DMA, pltpu.SemaphoreType.DMA,
            pltpu.SemaphoreType.DMA, pltpu.SemaphoreType.DMA,
            pltpu.SemaphoreType.DMA, pltpu.SemaphoreType.DMA,
        ],
    )
    def sc_k(batch_hbm, ent_hbm, rel_hbm, out_hbm,
             bidx_a, bidx_b,
             hidx_a, tidx_a, ridx_a, hidx_b, tidx_b, ridx_b,
             hrow_a, trow_a, rrow_a, hrow_b, trow_b, rrow_b,
             out_v,
             sh_a, st_a, sr_a, sh_b, st_b, sr_b):
        wid = lax.axis_index("s") * _NC + lax.axis_index("c")
        wbase = wid * per_w
        lanes = lax.iota(jnp.int32, _L)
        lanes3 = lanes * 3
        lanes_row = lanes * _D

        bufs_a = (bidx_a, hidx_a, tidx_a, ridx_a, hrow_a, trow_a, rrow_a,
                  sh_a, st_a, sr_a)
        bufs_b = (bidx_b, hidx_b, tidx_b, ridx_b, hrow_b, trow_b, rrow_b,
                  sh_b, st_b, sr_b)

        def issue(c, bufs):
            bidx_v, hidx_v, tidx_v, ridx_v, hrow_v, trow_v, rrow_v, sh, st, sr = bufs
            base = wbase + c * K
            pltpu.sync_copy(batch_hbm.at[pl.ds(base * 3, 3 * K)], bidx_v)
            for g in range(K // _L):
                b0 = g * (3 * _L)
                hidx_v[pl.ds(g * _L, _L)] = plsc.load_gather(
                    bidx_v, [b0 + lanes3])
                tidx_v[pl.ds(g * _L, _L)] = plsc.load_gather(
                    bidx_v, [b0 + lanes3 + 1])
                ridx_v[pl.ds(g * _L, _L)] = plsc.load_gather(
                    bidx_v, [b0 + lanes3 + 2])
            pltpu.async_copy(ent_hbm.at[hidx_v], hrow_v, sh)
            pltpu.async_copy(ent_hbm.at[tidx_v], trow_v, st)
            pltpu.async_copy(rel_hbm.at[ridx_v], rrow_v, sr)

        def wait_and_compute(c, bufs):
            _, hidx_v, tidx_v, ridx_v, hrow_v, trow_v, rrow_v, sh, st, sr = bufs
            base = wbase + c * K
            pltpu.make_async_copy(ent_hbm.at[hidx_v], hrow_v, sh).wait()
            pltpu.make_async_copy(ent_hbm.at[tidx_v], trow_v, st).wait()
            pltpu.make_async_copy(rel_hbm.at[ridx_v], rrow_v, sr).wait()
            def group_body(g, carry):
                ids = g * _L + lanes
                acc = jnp.zeros((_L,), jnp.float32)
                for d in range(_D):
                    col = (lanes + d) & (_D - 1)
                    hv = plsc.load_gather(hrow_v, [ids, col])
                    tv = plsc.load_gather(trow_v, [ids, col])
                    rv = plsc.load_gather(rrow_v, [ids, col])
                    acc = acc + hv * tv * rv
                out_v[pl.ds(g * _L, _L)] = acc
                return carry

            lax.fori_loop(0, K // _L, group_body, 0)
            pltpu.sync_copy(out_v, out_hbm.at[pl.ds(base, K)])

        issue(0, bufs_a)

        def body2(c2, carry):
            c = 2 * c2
            issue(c + 1, bufs_b)
            wait_and_compute(c, bufs_a)
            issue(c + 2, bufs_a)
            wait_and_compute(c + 1, bufs_b)
            return carry

        lax.fori_loop(0, half, body2, 0)
        wait_and_compute(n_chunks - 1, bufs_a)

    return sc_k


def kernel(graph, batch, entity_emb, relation_emb):
    B, C, _ = batch.shape
    n = B * C
    n_ent = entity_emb.shape[0]
    flat = jnp.minimum(batch.reshape(n * 3).astype(jnp.int32), n_ent - 1)
    sc_k = _build_sc_kernel(n)
    out = sc_k(flat, entity_emb, relation_emb)
    return out.reshape(B, C)


# speedup vs baseline: 2.4638x; 1.1481x over previous
import functools

import jax
import jax.numpy as jnp
from jax import lax
from jax.experimental import pallas as pl
from jax.experimental.pallas import tpu as pltpu
from jax.experimental.pallas import tpu_sc as plsc

_NC = 2
_NS = 16
_NW = _NC * _NS
_D = 64
_L = 16


def _build_sc_kernel(n_items):
    per_w = n_items // _NW
    K = 256
    n_chunks = per_w // K
    half = (n_chunks - 1) // 2

    row_buf = pltpu.VMEM((K, _D), jnp.float32)
    idx_buf = pltpu.VMEM((K,), jnp.int32)

    @functools.partial(
        pl.kernel,
        out_type=jax.ShapeDtypeStruct((n_items,), jnp.float32),
        mesh=plsc.VectorSubcoreMesh(core_axis_name="c", subcore_axis_name="s"),
        compiler_params=pltpu.CompilerParams(
            needs_layout_passes=False, use_tc_tiling_on_sc=False),
        scratch_types=[
            idx_buf, idx_buf, idx_buf,
            idx_buf, idx_buf, idx_buf,
            row_buf, row_buf, row_buf,
            row_buf, row_buf, row_buf,
            pltpu.VMEM((K,), jnp.float32),
            pltpu.SemaphoreType.DMA, pltpu.SemaphoreType.DMA,
            pltpu.SemaphoreType.DMA, pltpu.SemaphoreType.DMA,
            pltpu.SemaphoreType.DMA, pltpu.SemaphoreType.DMA,
        ],
    )
    def sc_k(idx_hbm, ent_hbm, rel_hbm, out_hbm,
             hidx_a, tidx_a, ridx_a, hidx_b, tidx_b, ridx_b,
             hrow_a, trow_a, rrow_a, hrow_b, trow_b, rrow_b,
             out_v,
             sh_a, st_a, sr_a, sh_b, st_b, sr_b):
        wid = lax.axis_index("s") * _NC + lax.axis_index("c")
        wbase = wid * per_w
        lanes = lax.iota(jnp.int32, _L)

        bufs_a = (hidx_a, tidx_a, ridx_a, hrow_a, trow_a, rrow_a,
                  sh_a, st_a, sr_a)
        bufs_b = (hidx_b, tidx_b, ridx_b, hrow_b, trow_b, rrow_b,
                  sh_b, st_b, sr_b)

        def issue(c, bufs):
            hidx_v, tidx_v, ridx_v, hrow_v, trow_v, rrow_v, sh, st, sr = bufs
            base = wbase + c * K
            pltpu.sync_copy(idx_hbm.at[pl.ds(base, K)], hidx_v)
            pltpu.sync_copy(idx_hbm.at[pl.ds(n_items + base, K)], tidx_v)
            pltpu.sync_copy(idx_hbm.at[pl.ds(2 * n_items + base, K)], ridx_v)
            pltpu.async_copy(ent_hbm.at[hidx_v], hrow_v, sh)
            pltpu.async_copy(ent_hbm.at[tidx_v], trow_v, st)
            pltpu.async_copy(rel_hbm.at[ridx_v], rrow_v, sr)

        def wait_and_compute(c, bufs):
            hidx_v, tidx_v, ridx_v, hrow_v, trow_v, rrow_v, sh, st, sr = bufs
            base = wbase + c * K
            pltpu.make_async_copy(ent_hbm.at[hidx_v], hrow_v, sh).wait()
            pltpu.make_async_copy(ent_hbm.at[tidx_v], trow_v, st).wait()
            pltpu.make_async_copy(rel_hbm.at[ridx_v], rrow_v, sr).wait()

            def group_body(g, carry):
                ids = g * _L + lanes
                acc = jnp.zeros((_L,), jnp.float32)
                for d in range(_D):
                    col = (lanes + d) & (_D - 1)
                    hv = plsc.load_gather(hrow_v, [ids, col])
                    tv = plsc.load_gather(trow_v, [ids, col])
                    rv = plsc.load_gather(rrow_v, [ids, col])
                    acc = acc + hv * tv * rv
                out_v[pl.ds(g * _L, _L)] = acc
                return carry

            lax.fori_loop(0, K // _L, group_body, 0)
            pltpu.sync_copy(out_v, out_hbm.at[pl.ds(base, K)])

        issue(0, bufs_a)

        def body2(c2, carry):
            c = 2 * c2
            issue(c + 1, bufs_b)
            wait_and_compute(c, bufs_a)
            issue(c + 2, bufs_a)
            wait_and_compute(c + 1, bufs_b)
            return carry

        lax.fori_loop(0, half, body2, 0)
        wait_and_compute(n_chunks - 1, bufs_a)

    return sc_k


def kernel(graph, batch, entity_emb, relation_emb):
    B, C, _ = batch.shape
    n = B * C
    n_ent = entity_emb.shape[0]
    planar = jnp.minimum(batch.transpose(2, 1, 0).reshape(3 * n), n_ent - 1) * 2
    ent = jnp.pad(entity_emb, ((0, 0), (0, _D))).reshape(2 * n_ent, _D)
    n_rel = relation_emb.shape[0]
    rel = jnp.pad(relation_emb, ((0, 0), (0, _D))).reshape(2 * n_rel, _D)
    sc_k = _build_sc_kernel(n)
    out = sc_k(planar, ent, rel)
    return out.reshape(C, B).transpose(1, 0)
